# Initial kernel scaffold; baseline (speedup 1.0000x reference)
import jax
import jax.numpy as jnp
from jax.experimental import pallas as pl


def kernel(x, num_nodes, edge_index, edge_attr, mask, W_edge, b_edge, W1, b1, W2, b2, ln1_g, ln1_b, ln2_g, ln2_b):
    raise NotImplementedError("write your pallas kernel here")


# speedup vs baseline: 6.7963x; 6.7963x over previous
import jax
import jax.numpy as jnp
from jax import lax
from jax.experimental import pallas as pl
from jax.experimental.pallas import tpu as pltpu
from jax.experimental.pallas import tpu_sc as plsc

N = 100000
E = 3200000
D = 16
NODE_DIM = 6

NROWS = 25600
E_PAD = NROWS * 128
PAD = E_PAD - E
PER_W = NROWS // 32
SC_NK = 8
SC_CHUNKS = PER_W // SC_NK
G_NK = 8
G_CHUNKS = PER_W // G_NK
N_PAD = 100352
N_PER_TILE = N_PAD // 16
N_STAGE = N_PER_TILE // 8

BE = 6400
BN = 2000

_SC_MESH = plsc.VectorSubcoreMesh(core_axis_name="c", subcore_axis_name="s")
_SC_PARAMS = pltpu.CompilerParams(use_tc_tiling_on_sc=False)


BE8 = 800
E8 = E // 8
E8_PAD = E_PAD // 8
E16 = E // 16
E16_PAD = E_PAD // 16


def _contrib_body(a8_ref, m8in_ref, m16in_ref, exg_ref, ex2_ref, c8_ref,
                  m8o_ref):
    i = pl.program_id(0)
    nb_real = E8 // BE8
    a = a8_ref[...]
    rows = lax.broadcasted_iota(jnp.int32, (BE8, 1), 0) + i * BE8
    valid = rows < E8
    m16 = jnp.dot(m8in_ref[...], exg_ref[...],
                  preferred_element_type=jnp.float32,
                  precision=lax.Precision.HIGHEST)
    c8_ref[...] = jnp.where(valid, a * m16, 0.0)
    rows2 = lax.broadcasted_iota(jnp.int32, (BE8 // 2, 1), 0) + i * (BE8 // 2)
    valid2 = rows2 < E16
    m8r = jnp.dot(m16in_ref[...], ex2_ref[...],
                  preferred_element_type=jnp.float32,
                  precision=lax.Precision.HIGHEST)
    m8o_ref[...] = jnp.where(valid2, m8r, 0.0)


def _contrib_call(a8, m8in, m16in, exg, ex2):
    nb_real = E8 // BE8
    grid = E8_PAD // BE8
    full = lambda i: (0, 0)
    return pl.pallas_call(
        _contrib_body,
        grid=(grid,),
        in_specs=[
            pl.BlockSpec((BE8, 128), lambda i: (jnp.minimum(i, 499), 0)),
            pl.BlockSpec((BE8, 8), lambda i: (jnp.minimum(i, 499), 0)),
            pl.BlockSpec((BE8 // 2, 16), lambda i: (jnp.minimum(i, 499), 0)),
            pl.BlockSpec((8, 128), full),
            pl.BlockSpec((16, 128), full),
        ],
        out_specs=[
            pl.BlockSpec((BE8, 128), lambda i: (i, 0)),
            pl.BlockSpec((BE8 // 2, 128), lambda i: (i, 0)),
        ],
        out_shape=[
            jax.ShapeDtypeStruct((E8_PAD, 128), jnp.float32),
            jax.ShapeDtypeStruct((E16_PAD, 128), jnp.float32),
        ],
    )(a8, m8in, m16in, exg, ex2)


def _scatter_attr_body(col2d, contrib, zeros, p0, p1, acc, contrib_v, idx_v,
                       sem):
    cid = lax.axis_index("c")
    sid = lax.axis_index("s")
    stage = contrib_v.at[pl.ds(0, N_STAGE)]

    for r in range(8):
        off = sid * N_PER_TILE + r * N_STAGE
        pltpu.sync_copy(zeros.at[pl.ds(off, N_STAGE)], stage)
        pltpu.sync_copy(stage, acc.at[pl.ds(off, N_STAGE)])
    plsc.subcore_barrier()

    base = cid * (16 * PER_W) + sid * PER_W

    def chunk(t, carry):
        r0 = base + t * SC_NK
        pltpu.sync_copy(col2d.at[pl.ds(r0, SC_NK)], idx_v)
        pltpu.sync_copy(contrib.at[pl.ds(r0 * 128, SC_NK * 128)], contrib_v)
        descs = []
        for j in range(SC_NK):
            descs.append(
                pltpu.async_copy(
                    contrib_v.at[pl.ds(j * 128, 128)],
                    acc.at[idx_v.at[j]],
                    sem,
                    add=True,
                ))
        for d in descs:
            d.wait()
        return carry

    lax.fori_loop(0, SC_CHUNKS, chunk, 0)
    plsc.subcore_barrier()

    for r in range(8):
        off = sid * N_PER_TILE + r * N_STAGE
        pltpu.sync_copy(acc.at[pl.ds(off, N_STAGE)], stage)

        @pl.when(cid == 0)
        def _():
            pltpu.sync_copy(stage, p0.at[pl.ds(off, N_STAGE)])

        @pl.when(cid == 1)
        def _():
            pltpu.sync_copy(stage, p1.at[pl.ds(off, N_STAGE)])


_scatter_attr_call = pl.kernel(
    _scatter_attr_body,
    out_type=[
        jax.ShapeDtypeStruct((N_PAD, D), jnp.float32),
        jax.ShapeDtypeStruct((N_PAD, D), jnp.float32),
    ],
    mesh=_SC_MESH,
    compiler_params=_SC_PARAMS,
    scratch_types=[
        pltpu.VMEM_SHARED((N_PAD, D), jnp.float32),
        pltpu.VMEM((SC_NK * 128, D), jnp.float32),
        pltpu.VMEM((SC_NK, 128), jnp.int32),
        pltpu.SemaphoreType.DMA,
    ],
)


MK = 16
M_CHUNKS = PER_W // MK


def _scatter_mask_body(col2d, mask2d, zeros, q0, q1, acc, mask_v, idx_v, sem):
    cid = lax.axis_index("c")
    sid = lax.axis_index("s")
    stage = mask_v.at[pl.ds(0, N_STAGE)]

    for r in range(8):
        off = sid * N_PER_TILE + r * N_STAGE
        pltpu.sync_copy(zeros.at[pl.ds(off, N_STAGE)], stage)
        pltpu.sync_copy(stage, acc.at[pl.ds(off, N_STAGE)])
    plsc.subcore_barrier()

    base = cid * (16 * PER_W) + sid * PER_W

    def chunk(t, carry):
        r0 = base + t * MK
        pltpu.sync_copy(col2d.at[pl.ds(r0, MK)], idx_v)
        pltpu.sync_copy(mask2d.at[pl.ds(r0 * 128, MK * 128)], mask_v)
        descs = []
        for j in range(MK):
            descs.append(
                pltpu.async_copy(
                    mask_v.at[pl.ds(j * 128, 128)],
                    acc.at[idx_v.at[j]],
                    sem,
                    add=True,
                ))
        for d in descs:
            d.wait()
        return carry

    lax.fori_loop(0, M_CHUNKS, chunk, 0)
    plsc.subcore_barrier()

    for r in range(8):
        off = sid * N_PER_TILE + r * N_STAGE
        pltpu.sync_copy(acc.at[pl.ds(off, N_STAGE)], stage)

        @pl.when(cid == 0)
        def _():
            pltpu.sync_copy(stage, q0.at[pl.ds(off, N_STAGE)])

        @pl.when(cid == 1)
        def _():
            pltpu.sync_copy(stage, q1.at[pl.ds(off, N_STAGE)])


_scatter_mask_call = pl.kernel(
    _scatter_mask_body,
    out_type=[
        jax.ShapeDtypeStruct((N_PAD, 8), jnp.float32),
        jax.ShapeDtypeStruct((N_PAD, 8), jnp.float32),
    ],
    mesh=_SC_MESH,
    compiler_params=_SC_PARAMS,
    scratch_types=[
        pltpu.VMEM_SHARED((N_PAD, 8), jnp.float32),
        pltpu.VMEM((MK * 128, 8), jnp.float32),
        pltpu.VMEM((MK, 128), jnp.int32),
        pltpu.SemaphoreType.DMA,
    ],
)


def _node_body(p0_ref, p1_ref, q0_ref, q1_ref, x_ref, wrt_ref, wct_ref,
               rep_ref, pr_ref, pc_ref):
    tot = p0_ref[...] + p1_ref[...]
    den = q0_ref[:, 0:1] + q1_ref[:, 0:1] + 1.0
    rep16 = tot / den
    rep = jnp.concatenate([rep16, x_ref[...]], axis=1)
    rep_ref[...] = rep
    pr_ref[...] = jnp.dot(rep, wrt_ref[...], preferred_element_type=jnp.float32,
                    precision=lax.Precision.HIGHEST)
    pc_ref[...] = jnp.dot(rep, wct_ref[...], preferred_element_type=jnp.float32,
                    precision=lax.Precision.HIGHEST)


def _node_call(p0, p1, q0, q1, x, wrt, wct):
    grid = N // BN
    full = lambda i: (0, 0)
    blk = lambda i: (i, 0)
    return pl.pallas_call(
        _node_body,
        grid=(grid,),
        in_specs=[
            pl.BlockSpec((BN, D), blk),
            pl.BlockSpec((BN, D), blk),
            pl.BlockSpec((BN, 8), blk),
            pl.BlockSpec((BN, 8), blk),
            pl.BlockSpec((BN, NODE_DIM), blk),
            pl.BlockSpec((D + NODE_DIM, D), full),
            pl.BlockSpec((D + NODE_DIM, D), full),
        ],
        out_specs=[
            pl.BlockSpec((BN, D + NODE_DIM), blk),
            pl.BlockSpec((BN, D), blk),
            pl.BlockSpec((BN, D), blk),
        ],
        out_shape=[
            jax.ShapeDtypeStruct((N, D + NODE_DIM), jnp.float32),
            jax.ShapeDtypeStruct((N, D), jnp.float32),
            jax.ShapeDtypeStruct((N, D), jnp.float32),
        ],
    )(p0, p1, q0, q1, x, wrt, wct)


def _gather_body(row2d, col2d, pr, pc, gr, gc, idx_r, idx_c, rows_r, rows_c,
                 sem):
    cid = lax.axis_index("c")
    sid = lax.axis_index("s")
    base = cid * (16 * PER_W) + sid * PER_W

    def chunk(t, carry):
        r0 = base + t * G_NK
        pltpu.sync_copy(row2d.at[pl.ds(r0, G_NK)], idx_r)
        pltpu.sync_copy(col2d.at[pl.ds(r0, G_NK)], idx_c)
        descs = []
        for j in range(G_NK):
            descs.append(
                pltpu.async_copy(pr.at[idx_r.at[j]],
                                 rows_r.at[pl.ds(j * 128, 128)], sem))
            descs.append(
                pltpu.async_copy(pc.at[idx_c.at[j]],
                                 rows_c.at[pl.ds(j * 128, 128)], sem))
        for d in descs:
            d.wait()
        pltpu.sync_copy(rows_r, gr.at[pl.ds(r0 * 128, G_NK * 128)])
        pltpu.sync_copy(rows_c, gc.at[pl.ds(r0 * 128, G_NK * 128)])
        return carry

    lax.fori_loop(0, G_CHUNKS, chunk, 0)


_gather_call = pl.kernel(
    _gather_body,
    out_type=[
        jax.ShapeDtypeStruct((E_PAD, D), jnp.float32),
        jax.ShapeDtypeStruct((E_PAD, D), jnp.float32),
    ],
    mesh=_SC_MESH,
    compiler_params=_SC_PARAMS,
    scratch_types=[
        pltpu.VMEM((G_NK, 128), jnp.int32),
        pltpu.VMEM((G_NK, 128), jnp.int32),
        pltpu.VMEM((G_NK * 128, D), jnp.float32),
        pltpu.VMEM((G_NK * 128, D), jnp.float32),
        pltpu.SemaphoreType.DMA,
    ],
)


def _mlp_body(a8_ref, gr8_ref, gc8_ref, watbd_ref, mmat_ref, w1bd_ref,
              w2bd_ref, be_ref, b1_ref, b2_ref, g1_ref, bb1_ref, g2_ref,
              bb2_ref, out_ref):
    dot = lambda x, w: jnp.dot(x, w, preferred_element_type=jnp.float32)
    a = a8_ref[...]
    h0 = dot(a, watbd_ref[...]) + gr8_ref[...] + gc8_ref[...] + be_ref[...]
    mm = mmat_ref[...]
    d1 = h0 - dot(h0, mm)
    var1 = dot(d1 * d1, mm)
    h = d1 * lax.rsqrt(var1 + 1e-5) * g1_ref[...] + bb1_ref[...]
    ff = jnp.maximum(dot(h, w1bd_ref[...]) + b1_ref[...], 0.0)
    s = h + dot(ff, w2bd_ref[...]) + b2_ref[...]
    d2 = s - dot(s, mm)
    var2 = dot(d2 * d2, mm)
    out_ref[...] = d2 * lax.rsqrt(var2 + 1e-5) * g2_ref[...] + bb2_ref[...]


def _mlp_call(a8, gr8, gc8, watbd, mmat, w1bd, w2bd, be, b1, b2, g1, bb1, g2,
              bb2):
    grid = E8 // BE8
    full = lambda i: (0, 0)
    blk = lambda i: (i, 0)
    return pl.pallas_call(
        _mlp_body,
        grid=(grid,),
        in_specs=[
            pl.BlockSpec((BE8, 128), blk),
            pl.BlockSpec((BE8, 128), blk),
            pl.BlockSpec((BE8, 128), blk),
            pl.BlockSpec((128, 128), full),
            pl.BlockSpec((128, 128), full),
            pl.BlockSpec((128, 256), full),
            pl.BlockSpec((256, 128), full),
            pl.BlockSpec((1, 128), full),
            pl.BlockSpec((1, 256), full),
            pl.BlockSpec((1, 128), full),
            pl.BlockSpec((1, 128), full),
            pl.BlockSpec((1, 128), full),
            pl.BlockSpec((1, 128), full),
            pl.BlockSpec((1, 128), full),
        ],
        out_specs=pl.BlockSpec((BE8, 128), blk),
        out_shape=jax.ShapeDtypeStruct((E8, 128), jnp.float32),
    )(a8, gr8, gc8, watbd, mmat, w1bd, w2bd, be, b1, b2, g1, bb1, g2, bb2)


def kernel(x, num_nodes, edge_index, edge_attr, mask, W_edge, b_edge, W1, b1,
           W2, b2, ln1_g, ln1_b, ln2_g, ln2_b):
    del num_nodes
    row = edge_index[0]
    col = edge_index[1]
    f32 = jnp.float32

    wrt = W_edge[:, :D + NODE_DIM].T
    wct = W_edge[:, D + NODE_DIM:2 * (D + NODE_DIM)].T
    wat = W_edge[:, 2 * (D + NODE_DIM):].T + jnp.eye(D, dtype=f32)
    eye8 = jnp.eye(8, dtype=f32)
    watbd = jnp.kron(eye8, wat)
    w1bd = jnp.kron(eye8, W1.T)
    w2bd = jnp.kron(eye8, W2.T)
    mmat = jnp.kron(eye8, jnp.full((D, D), 1.0 / D, f32))
    exg = jnp.kron(eye8, jnp.ones((1, D), f32))
    ex2 = jnp.kron(jnp.eye(D, dtype=f32), jnp.eye(8, dtype=f32)[0:1])
    tile8 = lambda v: jnp.tile(v, 8).reshape(1, -1)

    padv = lax.iota(jnp.int32, PAD)
    colp = jnp.concatenate([col, padv]).reshape(NROWS, 128)
    rowp = jnp.concatenate([row, padv]).reshape(NROWS, 128)

    a8 = edge_attr.reshape(E8, 128)
    contrib8, mask8 = _contrib_call(a8, mask.reshape(E8, 8),
                                    mask.reshape(E16, 16), exg, ex2)
    contrib = contrib8.reshape(E_PAD, D)
    zeros16 = jnp.zeros((N_PAD, D), f32)
    zeros8 = jnp.zeros((N_PAD, 8), f32)
    p0, p1 = _scatter_attr_call(colp, contrib, zeros16)
    q0, q1 = _scatter_mask_call(colp, mask8.reshape(E_PAD, 8), zeros8)
    node_rep, pr, pc = _node_call(p0, p1, q0, q1, x, wrt, wct)
    gr, gc = _gather_call(rowp, colp, pr, pc)
    out8 = _mlp_call(a8, gr.reshape(E8_PAD, 128),
                     gc.reshape(E8_PAD, 128), watbd, mmat, w1bd, w2bd,
                     tile8(b_edge), tile8(b1), tile8(b2), tile8(ln1_g),
                     tile8(ln1_b), tile8(ln2_g), tile8(ln2_b))
    return node_rep, out8.reshape(E, D)


# speedup vs baseline: 7.3172x; 1.0766x over previous
import jax
import jax.numpy as jnp
from jax import lax
from jax.experimental import pallas as pl
from jax.experimental.pallas import tpu as pltpu
from jax.experimental.pallas import tpu_sc as plsc

N = 100000
E = 3200000
D = 16
NODE_DIM = 6

NROWS = 25600
E_PAD = NROWS * 128
PAD = E_PAD - E
PER_W = NROWS // 32
SC_NK = 8
SC_CHUNKS = PER_W // SC_NK
G_NK = 8
G_CHUNKS = PER_W // G_NK
N_PAD = 100352
N_PER_TILE = N_PAD // 16
N_STAGE = N_PER_TILE // 8

BE = 6400
BN = 2000

_SC_MESH = plsc.VectorSubcoreMesh(core_axis_name="c", subcore_axis_name="s")
_SC_PARAMS = pltpu.CompilerParams(use_tc_tiling_on_sc=False)


BE8 = 1600
E8 = E // 8
E8_PAD = E_PAD // 8
E16 = E // 16
E16_PAD = E_PAD // 16


def _contrib_body(a8_ref, m8in_ref, m16in_ref, exg_ref, ex2_ref, c8_ref,
                  m8o_ref):
    i = pl.program_id(0)
    nb_real = E8 // BE8
    a = a8_ref[...]
    rows = lax.broadcasted_iota(jnp.int32, (BE8, 1), 0) + i * BE8
    valid = rows < E8
    m16 = jnp.dot(m8in_ref[...], exg_ref[...],
                  preferred_element_type=jnp.float32,
                  precision=lax.Precision.HIGHEST)
    c8_ref[...] = jnp.where(valid, a * m16, 0.0)
    rows2 = lax.broadcasted_iota(jnp.int32, (BE8 // 2, 1), 0) + i * (BE8 // 2)
    valid2 = rows2 < E16
    m8r = jnp.dot(m16in_ref[...], ex2_ref[...],
                  preferred_element_type=jnp.float32,
                  precision=lax.Precision.HIGHEST)
    m8o_ref[...] = jnp.where(valid2, m8r, 0.0)


def _contrib_call(a8, m8in, m16in, exg, ex2):
    nb_real = E8 // BE8
    grid = E8_PAD // BE8
    full = lambda i: (0, 0)
    return pl.pallas_call(
        _contrib_body,
        grid=(grid,),
        in_specs=[
            pl.BlockSpec((BE8, 128), lambda i: (jnp.minimum(i, E8 // BE8 - 1), 0)),
            pl.BlockSpec((BE8, 8), lambda i: (jnp.minimum(i, E8 // BE8 - 1), 0)),
            pl.BlockSpec((BE8 // 2, 16), lambda i: (jnp.minimum(i, E8 // BE8 - 1), 0)),
            pl.BlockSpec((8, 128), full),
            pl.BlockSpec((16, 128), full),
        ],
        out_specs=[
            pl.BlockSpec((BE8, 128), lambda i: (i, 0)),
            pl.BlockSpec((BE8 // 2, 128), lambda i: (i, 0)),
        ],
        out_shape=[
            jax.ShapeDtypeStruct((E8_PAD, 128), jnp.float32),
            jax.ShapeDtypeStruct((E16_PAD, 128), jnp.float32),
        ],
    )(a8, m8in, m16in, exg, ex2)


def _scatter_attr_body(col2d, contrib, zeros, p0, p1, acc, contrib_v, idx_v,
                       sem):
    cid = lax.axis_index("c")
    sid = lax.axis_index("s")
    stage = contrib_v.at[pl.ds(0, N_STAGE)]

    for r in range(8):
        off = sid * N_PER_TILE + r * N_STAGE
        pltpu.sync_copy(zeros.at[pl.ds(off, N_STAGE)], stage)
        pltpu.sync_copy(stage, acc.at[pl.ds(off, N_STAGE)])
    plsc.subcore_barrier()

    base = cid * (16 * PER_W) + sid * PER_W

    def chunk(t, carry):
        r0 = base + t * SC_NK
        pltpu.sync_copy(col2d.at[pl.ds(r0, SC_NK)], idx_v)
        pltpu.sync_copy(contrib.at[pl.ds(r0 * 128, SC_NK * 128)], contrib_v)
        descs = []
        for j in range(SC_NK):
            descs.append(
                pltpu.async_copy(
                    contrib_v.at[pl.ds(j * 128, 128)],
                    acc.at[idx_v.at[j]],
                    sem,
                    add=True,
                ))
        for d in descs:
            d.wait()
        return carry

    lax.fori_loop(0, SC_CHUNKS, chunk, 0)
    plsc.subcore_barrier()

    for r in range(8):
        off = sid * N_PER_TILE + r * N_STAGE
        pltpu.sync_copy(acc.at[pl.ds(off, N_STAGE)], stage)

        @pl.when(cid == 0)
        def _():
            pltpu.sync_copy(stage, p0.at[pl.ds(off, N_STAGE)])

        @pl.when(cid == 1)
        def _():
            pltpu.sync_copy(stage, p1.at[pl.ds(off, N_STAGE)])


_scatter_attr_call = pl.kernel(
    _scatter_attr_body,
    out_type=[
        jax.ShapeDtypeStruct((N_PAD, D), jnp.float32),
        jax.ShapeDtypeStruct((N_PAD, D), jnp.float32),
    ],
    mesh=_SC_MESH,
    compiler_params=_SC_PARAMS,
    scratch_types=[
        pltpu.VMEM_SHARED((N_PAD, D), jnp.float32),
        pltpu.VMEM((SC_NK * 128, D), jnp.float32),
        pltpu.VMEM((SC_NK, 128), jnp.int32),
        pltpu.SemaphoreType.DMA,
    ],
)


MK = 16
M_CHUNKS = PER_W // MK


def _scatter_mask_body(col2d, mask2d, zeros, q0, q1, acc, mask_v, idx_v, sem):
    cid = lax.axis_index("c")
    sid = lax.axis_index("s")
    stage = mask_v.at[pl.ds(0, N_STAGE)]

    for r in range(8):
        off = sid * N_PER_TILE + r * N_STAGE
        pltpu.sync_copy(zeros.at[pl.ds(off, N_STAGE)], stage)
        pltpu.sync_copy(stage, acc.at[pl.ds(off, N_STAGE)])
    plsc.subcore_barrier()

    base = cid * (16 * PER_W) + sid * PER_W

    def chunk(t, carry):
        r0 = base + t * MK
        pltpu.sync_copy(col2d.at[pl.ds(r0, MK)], idx_v)
        pltpu.sync_copy(mask2d.at[pl.ds(r0 * 128, MK * 128)], mask_v)
        descs = []
        for j in range(MK):
            descs.append(
                pltpu.async_copy(
                    mask_v.at[pl.ds(j * 128, 128)],
                    acc.at[idx_v.at[j]],
                    sem,
                    add=True,
                ))
        for d in descs:
            d.wait()
        return carry

    lax.fori_loop(0, M_CHUNKS, chunk, 0)
    plsc.subcore_barrier()

    for r in range(8):
        off = sid * N_PER_TILE + r * N_STAGE
        pltpu.sync_copy(acc.at[pl.ds(off, N_STAGE)], stage)

        @pl.when(cid == 0)
        def _():
            pltpu.sync_copy(stage, q0.at[pl.ds(off, N_STAGE)])

        @pl.when(cid == 1)
        def _():
            pltpu.sync_copy(stage, q1.at[pl.ds(off, N_STAGE)])


_scatter_mask_call = pl.kernel(
    _scatter_mask_body,
    out_type=[
        jax.ShapeDtypeStruct((N_PAD, 8), jnp.float32),
        jax.ShapeDtypeStruct((N_PAD, 8), jnp.float32),
    ],
    mesh=_SC_MESH,
    compiler_params=_SC_PARAMS,
    scratch_types=[
        pltpu.VMEM_SHARED((N_PAD, 8), jnp.float32),
        pltpu.VMEM((MK * 128, 8), jnp.float32),
        pltpu.VMEM((MK, 128), jnp.int32),
        pltpu.SemaphoreType.DMA,
    ],
)


def _node_body(p0_ref, p1_ref, q0_ref, q1_ref, x_ref, wrt_ref, wct_ref,
               rep_ref, pr_ref, pc_ref):
    tot = p0_ref[...] + p1_ref[...]
    den = q0_ref[:, 0:1] + q1_ref[:, 0:1] + 1.0
    rep16 = tot / den
    rep = jnp.concatenate([rep16, x_ref[...]], axis=1)
    rep_ref[...] = rep
    pr_ref[...] = jnp.dot(rep, wrt_ref[...], preferred_element_type=jnp.float32,
                    precision=lax.Precision.HIGHEST)
    pc_ref[...] = jnp.dot(rep, wct_ref[...], preferred_element_type=jnp.float32,
                    precision=lax.Precision.HIGHEST)


def _node_call(p0, p1, q0, q1, x, wrt, wct):
    grid = N // BN
    full = lambda i: (0, 0)
    blk = lambda i: (i, 0)
    return pl.pallas_call(
        _node_body,
        grid=(grid,),
        in_specs=[
            pl.BlockSpec((BN, D), blk),
            pl.BlockSpec((BN, D), blk),
            pl.BlockSpec((BN, 8), blk),
            pl.BlockSpec((BN, 8), blk),
            pl.BlockSpec((BN, NODE_DIM), blk),
            pl.BlockSpec((D + NODE_DIM, D), full),
            pl.BlockSpec((D + NODE_DIM, D), full),
        ],
        out_specs=[
            pl.BlockSpec((BN, D + NODE_DIM), blk),
            pl.BlockSpec((BN, D), blk),
            pl.BlockSpec((BN, D), blk),
        ],
        out_shape=[
            jax.ShapeDtypeStruct((N, D + NODE_DIM), jnp.float32),
            jax.ShapeDtypeStruct((N, D), jnp.float32),
            jax.ShapeDtypeStruct((N, D), jnp.float32),
        ],
    )(p0, p1, q0, q1, x, wrt, wct)


def _gather_body(row2d, col2d, pr, pc, gr, gc, idx_r, idx_c, rows_r, rows_c,
                 sem):
    cid = lax.axis_index("c")
    sid = lax.axis_index("s")
    base = cid * (16 * PER_W) + sid * PER_W

    def chunk(t, carry):
        r0 = base + t * G_NK
        pltpu.sync_copy(row2d.at[pl.ds(r0, G_NK)], idx_r)
        pltpu.sync_copy(col2d.at[pl.ds(r0, G_NK)], idx_c)
        descs = []
        for j in range(G_NK):
            descs.append(
                pltpu.async_copy(pr.at[idx_r.at[j]],
                                 rows_r.at[pl.ds(j * 128, 128)], sem))
            descs.append(
                pltpu.async_copy(pc.at[idx_c.at[j]],
                                 rows_c.at[pl.ds(j * 128, 128)], sem))
        for d in descs:
            d.wait()
        pltpu.sync_copy(rows_r, gr.at[pl.ds(r0 * 128, G_NK * 128)])
        pltpu.sync_copy(rows_c, gc.at[pl.ds(r0 * 128, G_NK * 128)])
        return carry

    lax.fori_loop(0, G_CHUNKS, chunk, 0)


_gather_call = pl.kernel(
    _gather_body,
    out_type=[
        jax.ShapeDtypeStruct((E_PAD, D), jnp.float32),
        jax.ShapeDtypeStruct((E_PAD, D), jnp.float32),
    ],
    mesh=_SC_MESH,
    compiler_params=_SC_PARAMS,
    scratch_types=[
        pltpu.VMEM((G_NK, 128), jnp.int32),
        pltpu.VMEM((G_NK, 128), jnp.int32),
        pltpu.VMEM((G_NK * 128, D), jnp.float32),
        pltpu.VMEM((G_NK * 128, D), jnp.float32),
        pltpu.SemaphoreType.DMA,
    ],
)


def _mlp_body(a8_ref, gr8_ref, gc8_ref, watbd_ref, mmat_ref, w1bd_ref,
              w2bd_ref, be_ref, b1_ref, b2_ref, g1_ref, bb1_ref, g2_ref,
              bb2_ref, out_ref):
    dot = lambda x, w: jnp.dot(x, w, preferred_element_type=jnp.float32)
    a = a8_ref[...]
    h0 = dot(a, watbd_ref[...]) + gr8_ref[...] + gc8_ref[...] + be_ref[...]
    mm = mmat_ref[...]
    d1 = h0 - dot(h0, mm)
    var1 = dot(d1 * d1, mm)
    h = d1 * lax.rsqrt(var1 + 1e-5) * g1_ref[...] + bb1_ref[...]
    ff = jnp.maximum(dot(h, w1bd_ref[...]) + b1_ref[...], 0.0)
    s = h + dot(ff, w2bd_ref[...]) + b2_ref[...]
    d2 = s - dot(s, mm)
    var2 = dot(d2 * d2, mm)
    out_ref[...] = d2 * lax.rsqrt(var2 + 1e-5) * g2_ref[...] + bb2_ref[...]


def _mlp_call(a8, gr8, gc8, watbd, mmat, w1bd, w2bd, be, b1, b2, g1, bb1, g2,
              bb2):
    grid = E8 // BE8
    full = lambda i: (0, 0)
    blk = lambda i: (i, 0)
    return pl.pallas_call(
        _mlp_body,
        grid=(grid,),
        in_specs=[
            pl.BlockSpec((BE8, 128), blk),
            pl.BlockSpec((BE8, 128), blk),
            pl.BlockSpec((BE8, 128), blk),
            pl.BlockSpec((128, 128), full),
            pl.BlockSpec((128, 128), full),
            pl.BlockSpec((128, 256), full),
            pl.BlockSpec((256, 128), full),
            pl.BlockSpec((1, 128), full),
            pl.BlockSpec((1, 256), full),
            pl.BlockSpec((1, 128), full),
            pl.BlockSpec((1, 128), full),
            pl.BlockSpec((1, 128), full),
            pl.BlockSpec((1, 128), full),
            pl.BlockSpec((1, 128), full),
        ],
        out_specs=pl.BlockSpec((BE8, 128), blk),
        out_shape=jax.ShapeDtypeStruct((E8, 128), jnp.float32),
    )(a8, gr8, gc8, watbd, mmat, w1bd, w2bd, be, b1, b2, g1, bb1, g2, bb2)


def kernel(x, num_nodes, edge_index, edge_attr, mask, W_edge, b_edge, W1, b1,
           W2, b2, ln1_g, ln1_b, ln2_g, ln2_b):
    del num_nodes
    row = edge_index[0]
    col = edge_index[1]
    f32 = jnp.float32

    wrt = W_edge[:, :D + NODE_DIM].T
    wct = W_edge[:, D + NODE_DIM:2 * (D + NODE_DIM)].T
    wat = W_edge[:, 2 * (D + NODE_DIM):].T + jnp.eye(D, dtype=f32)
    eye8 = jnp.eye(8, dtype=f32)
    watbd = jnp.kron(eye8, wat)
    w1bd = jnp.kron(eye8, W1.T)
    w2bd = jnp.kron(eye8, W2.T)
    mmat = jnp.kron(eye8, jnp.full((D, D), 1.0 / D, f32))
    exg = jnp.kron(eye8, jnp.ones((1, D), f32))
    ex2 = jnp.kron(jnp.eye(D, dtype=f32), jnp.eye(8, dtype=f32)[0:1])
    tile8 = lambda v: jnp.tile(v, 8).reshape(1, -1)

    padv = lax.iota(jnp.int32, PAD)
    colp = jnp.concatenate([col, padv]).reshape(NROWS, 128)
    rowp = jnp.concatenate([row, padv]).reshape(NROWS, 128)

    a8 = edge_attr.reshape(E8, 128)
    contrib8, mask8 = _contrib_call(a8, mask.reshape(E8, 8),
                                    mask.reshape(E16, 16), exg, ex2)
    contrib = contrib8.reshape(E_PAD, D)
    zeros16 = jnp.zeros((N_PAD, D), f32)
    zeros8 = jnp.zeros((N_PAD, 8), f32)
    p0, p1 = _scatter_attr_call(colp, contrib, zeros16)
    q0, q1 = _scatter_mask_call(colp, mask8.reshape(E_PAD, 8), zeros8)
    node_rep, pr, pc = _node_call(p0, p1, q0, q1, x, wrt, wct)
    gr, gc = _gather_call(rowp, colp, pr, pc)
    out8 = _mlp_call(a8, gr.reshape(E8_PAD, 128),
                     gc.reshape(E8_PAD, 128), watbd, mmat, w1bd, w2bd,
                     tile8(b_edge), tile8(b1), tile8(b2), tile8(ln1_g),
                     tile8(ln1_b), tile8(ln2_g), tile8(ln2_b))
    return node_rep, out8.reshape(E, D)


# speedup vs baseline: 7.4639x; 1.0200x over previous
import jax
import jax.numpy as jnp
from jax import lax
from jax.experimental import pallas as pl
from jax.experimental.pallas import tpu as pltpu
from jax.experimental.pallas import tpu_sc as plsc

N = 100000
E = 3200000
D = 16
NODE_DIM = 6

NROWS = 25600
E_PAD = NROWS * 128
PAD = E_PAD - E
PER_W = NROWS // 32
SC_NK = 8
SC_CHUNKS = PER_W // SC_NK
G_NK = 16
G_CHUNKS = PER_W // G_NK
N_PAD = 100352
N_PER_TILE = N_PAD // 16
N_STAGE = N_PER_TILE // 8

BE = 6400
BN = 2000

_SC_MESH = plsc.VectorSubcoreMesh(core_axis_name="c", subcore_axis_name="s")
_SC_PARAMS = pltpu.CompilerParams(use_tc_tiling_on_sc=False)


BE8 = 1600
E8 = E // 8
E8_PAD = E_PAD // 8
E16 = E // 16
E16_PAD = E_PAD // 16


def _contrib_body(a8_ref, m8in_ref, m16in_ref, exg_ref, ex2_ref, c8_ref,
                  m8o_ref):
    i = pl.program_id(0)
    nb_real = E8 // BE8
    a = a8_ref[...]
    rows = lax.broadcasted_iota(jnp.int32, (BE8, 1), 0) + i * BE8
    valid = rows < E8
    m16 = jnp.dot(m8in_ref[...], exg_ref[...],
                  preferred_element_type=jnp.float32,
                  precision=lax.Precision.HIGHEST)
    c8_ref[...] = jnp.where(valid, a * m16, 0.0)
    rows2 = lax.broadcasted_iota(jnp.int32, (BE8 // 2, 1), 0) + i * (BE8 // 2)
    valid2 = rows2 < E16
    m8r = jnp.dot(m16in_ref[...], ex2_ref[...],
                  preferred_element_type=jnp.float32,
                  precision=lax.Precision.HIGHEST)
    m8o_ref[...] = jnp.where(valid2, m8r, 0.0)


def _contrib_call(a8, m8in, m16in, exg, ex2):
    nb_real = E8 // BE8
    grid = E8_PAD // BE8
    full = lambda i: (0, 0)
    return pl.pallas_call(
        _contrib_body,
        grid=(grid,),
        in_specs=[
            pl.BlockSpec((BE8, 128), lambda i: (jnp.minimum(i, E8 // BE8 - 1), 0)),
            pl.BlockSpec((BE8, 8), lambda i: (jnp.minimum(i, E8 // BE8 - 1), 0)),
            pl.BlockSpec((BE8 // 2, 16), lambda i: (jnp.minimum(i, E8 // BE8 - 1), 0)),
            pl.BlockSpec((8, 128), full),
            pl.BlockSpec((16, 128), full),
        ],
        out_specs=[
            pl.BlockSpec((BE8, 128), lambda i: (i, 0)),
            pl.BlockSpec((BE8 // 2, 128), lambda i: (i, 0)),
        ],
        out_shape=[
            jax.ShapeDtypeStruct((E8_PAD, 128), jnp.float32),
            jax.ShapeDtypeStruct((E16_PAD, 128), jnp.float32),
        ],
    )(a8, m8in, m16in, exg, ex2)


def _scatter_attr_body(col2d, contrib, zeros, p0, p1, acc, contrib_v, idx_v,
                       sem):
    cid = lax.axis_index("c")
    sid = lax.axis_index("s")
    stage = contrib_v.at[pl.ds(0, N_STAGE)]

    for r in range(8):
        off = sid * N_PER_TILE + r * N_STAGE
        pltpu.sync_copy(zeros.at[pl.ds(off, N_STAGE)], stage)
        pltpu.sync_copy(stage, acc.at[pl.ds(off, N_STAGE)])
    plsc.subcore_barrier()

    base = cid * (16 * PER_W) + sid * PER_W

    def chunk(t, carry):
        r0 = base + t * SC_NK
        pltpu.sync_copy(col2d.at[pl.ds(r0, SC_NK)], idx_v)
        pltpu.sync_copy(contrib.at[pl.ds(r0 * 128, SC_NK * 128)], contrib_v)
        descs = []
        for j in range(SC_NK):
            descs.append(
                pltpu.async_copy(
                    contrib_v.at[pl.ds(j * 128, 128)],
                    acc.at[idx_v.at[j]],
                    sem,
                    add=True,
                ))
        for d in descs:
            d.wait()
        return carry

    lax.fori_loop(0, SC_CHUNKS, chunk, 0)
    plsc.subcore_barrier()

    for r in range(8):
        off = sid * N_PER_TILE + r * N_STAGE
        pltpu.sync_copy(acc.at[pl.ds(off, N_STAGE)], stage)

        @pl.when(cid == 0)
        def _():
            pltpu.sync_copy(stage, p0.at[pl.ds(off, N_STAGE)])

        @pl.when(cid == 1)
        def _():
            pltpu.sync_copy(stage, p1.at[pl.ds(off, N_STAGE)])


_scatter_attr_call = pl.kernel(
    _scatter_attr_body,
    out_type=[
        jax.ShapeDtypeStruct((N_PAD, D), jnp.float32),
        jax.ShapeDtypeStruct((N_PAD, D), jnp.float32),
    ],
    mesh=_SC_MESH,
    compiler_params=_SC_PARAMS,
    scratch_types=[
        pltpu.VMEM_SHARED((N_PAD, D), jnp.float32),
        pltpu.VMEM((SC_NK * 128, D), jnp.float32),
        pltpu.VMEM((SC_NK, 128), jnp.int32),
        pltpu.SemaphoreType.DMA,
    ],
)


MK = 16
M_CHUNKS = PER_W // MK


def _scatter_mask_body(col2d, mask2d, zeros, q0, q1, acc, mask_v, idx_v, sem):
    cid = lax.axis_index("c")
    sid = lax.axis_index("s")
    stage = mask_v.at[pl.ds(0, N_STAGE)]

    for r in range(8):
        off = sid * N_PER_TILE + r * N_STAGE
        pltpu.sync_copy(zeros.at[pl.ds(off, N_STAGE)], stage)
        pltpu.sync_copy(stage, acc.at[pl.ds(off, N_STAGE)])
    plsc.subcore_barrier()

    base = cid * (16 * PER_W) + sid * PER_W

    def chunk(t, carry):
        r0 = base + t * MK
        pltpu.sync_copy(col2d.at[pl.ds(r0, MK)], idx_v)
        pltpu.sync_copy(mask2d.at[pl.ds(r0 * 128, MK * 128)], mask_v)
        descs = []
        for j in range(MK):
            descs.append(
                pltpu.async_copy(
                    mask_v.at[pl.ds(j * 128, 128)],
                    acc.at[idx_v.at[j]],
                    sem,
                    add=True,
                ))
        for d in descs:
            d.wait()
        return carry

    lax.fori_loop(0, M_CHUNKS, chunk, 0)
    plsc.subcore_barrier()

    for r in range(8):
        off = sid * N_PER_TILE + r * N_STAGE
        pltpu.sync_copy(acc.at[pl.ds(off, N_STAGE)], stage)

        @pl.when(cid == 0)
        def _():
            pltpu.sync_copy(stage, q0.at[pl.ds(off, N_STAGE)])

        @pl.when(cid == 1)
        def _():
            pltpu.sync_copy(stage, q1.at[pl.ds(off, N_STAGE)])


_scatter_mask_call = pl.kernel(
    _scatter_mask_body,
    out_type=[
        jax.ShapeDtypeStruct((N_PAD, 8), jnp.float32),
        jax.ShapeDtypeStruct((N_PAD, 8), jnp.float32),
    ],
    mesh=_SC_MESH,
    compiler_params=_SC_PARAMS,
    scratch_types=[
        pltpu.VMEM_SHARED((N_PAD, 8), jnp.float32),
        pltpu.VMEM((MK * 128, 8), jnp.float32),
        pltpu.VMEM((MK, 128), jnp.int32),
        pltpu.SemaphoreType.DMA,
    ],
)


def _node_body(p0_ref, p1_ref, q0_ref, q1_ref, x_ref, wrt_ref, wct_ref,
               rep_ref, pr_ref, pc_ref):
    tot = p0_ref[...] + p1_ref[...]
    den = q0_ref[:, 0:1] + q1_ref[:, 0:1] + 1.0
    rep16 = tot / den
    rep = jnp.concatenate([rep16, x_ref[...]], axis=1)
    rep_ref[...] = rep
    pr_ref[...] = jnp.dot(rep, wrt_ref[...], preferred_element_type=jnp.float32,
                    precision=lax.Precision.HIGHEST)
    pc_ref[...] = jnp.dot(rep, wct_ref[...], preferred_element_type=jnp.float32,
                    precision=lax.Precision.HIGHEST)


def _node_call(p0, p1, q0, q1, x, wrt, wct):
    grid = N // BN
    full = lambda i: (0, 0)
    blk = lambda i: (i, 0)
    return pl.pallas_call(
        _node_body,
        grid=(grid,),
        in_specs=[
            pl.BlockSpec((BN, D), blk),
            pl.BlockSpec((BN, D), blk),
            pl.BlockSpec((BN, 8), blk),
            pl.BlockSpec((BN, 8), blk),
            pl.BlockSpec((BN, NODE_DIM), blk),
            pl.BlockSpec((D + NODE_DIM, D), full),
            pl.BlockSpec((D + NODE_DIM, D), full),
        ],
        out_specs=[
            pl.BlockSpec((BN, D + NODE_DIM), blk),
            pl.BlockSpec((BN, D), blk),
            pl.BlockSpec((BN, D), blk),
        ],
        out_shape=[
            jax.ShapeDtypeStruct((N, D + NODE_DIM), jnp.float32),
            jax.ShapeDtypeStruct((N, D), jnp.float32),
            jax.ShapeDtypeStruct((N, D), jnp.float32),
        ],
    )(p0, p1, q0, q1, x, wrt, wct)


def _gather_body(row2d, col2d, pr, pc, gr, gc, idx_r, idx_c, rows_r, rows_c,
                 sem):
    cid = lax.axis_index("c")
    sid = lax.axis_index("s")
    base = cid * (16 * PER_W) + sid * PER_W

    def chunk(t, carry):
        r0 = base + t * G_NK
        pltpu.sync_copy(row2d.at[pl.ds(r0, G_NK)], idx_r)
        pltpu.sync_copy(col2d.at[pl.ds(r0, G_NK)], idx_c)
        descs = []
        for j in range(G_NK):
            descs.append(
                pltpu.async_copy(pr.at[idx_r.at[j]],
                                 rows_r.at[pl.ds(j * 128, 128)], sem))
            descs.append(
                pltpu.async_copy(pc.at[idx_c.at[j]],
                                 rows_c.at[pl.ds(j * 128, 128)], sem))
        for d in descs:
            d.wait()
        pltpu.sync_copy(rows_r, gr.at[pl.ds(r0 * 128, G_NK * 128)])
        pltpu.sync_copy(rows_c, gc.at[pl.ds(r0 * 128, G_NK * 128)])
        return carry

    lax.fori_loop(0, G_CHUNKS, chunk, 0)


_gather_call = pl.kernel(
    _gather_body,
    out_type=[
        jax.ShapeDtypeStruct((E_PAD, D), jnp.float32),
        jax.ShapeDtypeStruct((E_PAD, D), jnp.float32),
    ],
    mesh=_SC_MESH,
    compiler_params=_SC_PARAMS,
    scratch_types=[
        pltpu.VMEM((G_NK, 128), jnp.int32),
        pltpu.VMEM((G_NK, 128), jnp.int32),
        pltpu.VMEM((G_NK * 128, D), jnp.float32),
        pltpu.VMEM((G_NK * 128, D), jnp.float32),
        pltpu.SemaphoreType.DMA,
    ],
)


def _mlp_body(a8_ref, gr8_ref, gc8_ref, watbd_ref, mmat_ref, w1bd_ref,
              w2bd_ref, be_ref, b1_ref, b2_ref, g1_ref, bb1_ref, g2_ref,
              bb2_ref, out_ref):
    dot = lambda x, w: jnp.dot(x, w, preferred_element_type=jnp.float32)
    a = a8_ref[...]
    h0 = dot(a, watbd_ref[...]) + gr8_ref[...] + gc8_ref[...] + be_ref[...]
    mm = mmat_ref[...]
    d1 = h0 - dot(h0, mm)
    var1 = dot(d1 * d1, mm)
    h = d1 * lax.rsqrt(var1 + 1e-5) * g1_ref[...] + bb1_ref[...]
    ff = jnp.maximum(dot(h, w1bd_ref[...]) + b1_ref[...], 0.0)
    s = h + dot(ff, w2bd_ref[...]) + b2_ref[...]
    d2 = s - dot(s, mm)
    var2 = dot(d2 * d2, mm)
    out_ref[...] = d2 * lax.rsqrt(var2 + 1e-5) * g2_ref[...] + bb2_ref[...]


def _mlp_call(a8, gr8, gc8, watbd, mmat, w1bd, w2bd, be, b1, b2, g1, bb1, g2,
              bb2):
    grid = E8 // BE8
    full = lambda i: (0, 0)
    blk = lambda i: (i, 0)
    return pl.pallas_call(
        _mlp_body,
        grid=(grid,),
        in_specs=[
            pl.BlockSpec((BE8, 128), blk),
            pl.BlockSpec((BE8, 128), blk),
            pl.BlockSpec((BE8, 128), blk),
            pl.BlockSpec((128, 128), full),
            pl.BlockSpec((128, 128), full),
            pl.BlockSpec((128, 256), full),
            pl.BlockSpec((256, 128), full),
            pl.BlockSpec((1, 128), full),
            pl.BlockSpec((1, 256), full),
            pl.BlockSpec((1, 128), full),
            pl.BlockSpec((1, 128), full),
            pl.BlockSpec((1, 128), full),
            pl.BlockSpec((1, 128), full),
            pl.BlockSpec((1, 128), full),
        ],
        out_specs=pl.BlockSpec((BE8, 128), blk),
        out_shape=jax.ShapeDtypeStruct((E8, 128), jnp.float32),
    )(a8, gr8, gc8, watbd, mmat, w1bd, w2bd, be, b1, b2, g1, bb1, g2, bb2)


def kernel(x, num_nodes, edge_index, edge_attr, mask, W_edge, b_edge, W1, b1,
           W2, b2, ln1_g, ln1_b, ln2_g, ln2_b):
    del num_nodes
    row = edge_index[0]
    col = edge_index[1]
    f32 = jnp.float32

    wrt = W_edge[:, :D + NODE_DIM].T
    wct = W_edge[:, D + NODE_DIM:2 * (D + NODE_DIM)].T
    wat = W_edge[:, 2 * (D + NODE_DIM):].T + jnp.eye(D, dtype=f32)
    eye8 = jnp.eye(8, dtype=f32)
    watbd = jnp.kron(eye8, wat)
    w1bd = jnp.kron(eye8, W1.T)
    w2bd = jnp.kron(eye8, W2.T)
    mmat = jnp.kron(eye8, jnp.full((D, D), 1.0 / D, f32))
    exg = jnp.kron(eye8, jnp.ones((1, D), f32))
    ex2 = jnp.kron(jnp.eye(D, dtype=f32), jnp.eye(8, dtype=f32)[0:1])
    tile8 = lambda v: jnp.tile(v, 8).reshape(1, -1)

    padv = lax.iota(jnp.int32, PAD)
    colp = jnp.concatenate([col, padv]).reshape(NROWS, 128)
    rowp = jnp.concatenate([row, padv]).reshape(NROWS, 128)

    a8 = edge_attr.reshape(E8, 128)
    contrib8, mask8 = _contrib_call(a8, mask.reshape(E8, 8),
                                    mask.reshape(E16, 16), exg, ex2)
    contrib = contrib8.reshape(E_PAD, D)
    zeros16 = jnp.zeros((N_PAD, D), f32)
    zeros8 = jnp.zeros((N_PAD, 8), f32)
    p0, p1 = _scatter_attr_call(colp, contrib, zeros16)
    q0, q1 = _scatter_mask_call(colp, mask8.reshape(E_PAD, 8), zeros8)
    node_rep, pr, pc = _node_call(p0, p1, q0, q1, x, wrt, wct)
    gr, gc = _gather_call(rowp, colp, pr, pc)
    out8 = _mlp_call(a8, gr.reshape(E8_PAD, 128),
                     gc.reshape(E8_PAD, 128), watbd, mmat, w1bd, w2bd,
                     tile8(b_edge), tile8(b1), tile8(b2), tile8(ln1_g),
                     tile8(ln1_b), tile8(ln2_g), tile8(ln2_b))
    return node_rep, out8.reshape(E, D)


# speedup vs baseline: 7.7474x; 1.0380x over previous
import jax
import jax.numpy as jnp
from jax import lax
from jax.experimental import pallas as pl
from jax.experimental.pallas import tpu as pltpu
from jax.experimental.pallas import tpu_sc as plsc

N = 100000
E = 3200000
D = 16
NODE_DIM = 6

NROWS = 25600
E_PAD = NROWS * 128
PAD = E_PAD - E
PER_W = NROWS // 32
SC_NK = 8
SC_CHUNKS = PER_W // SC_NK
G_NK = 16
G_CHUNKS = PER_W // G_NK
N_PAD = 100352
N_PER_TILE = N_PAD // 16
N_STAGE = N_PER_TILE // 8

BE = 6400
BN = 2000

_SC_MESH = plsc.VectorSubcoreMesh(core_axis_name="c", subcore_axis_name="s")
_SC_PARAMS = pltpu.CompilerParams(use_tc_tiling_on_sc=False)


BE8 = 3200
E8 = E // 8
E8_PAD = E_PAD // 8
E16 = E // 16
E16_PAD = E_PAD // 16


def _contrib_body(a8_ref, m8in_ref, m16in_ref, exg_ref, ex2_ref, c8_ref,
                  m8o_ref):
    i = pl.program_id(0)
    nb_real = E8 // BE8
    a = a8_ref[...]
    rows = lax.broadcasted_iota(jnp.int32, (BE8, 1), 0) + i * BE8
    valid = rows < E8
    m16 = jnp.dot(m8in_ref[...], exg_ref[...],
                  preferred_element_type=jnp.float32,
                  precision=lax.Precision.HIGHEST)
    c8_ref[...] = jnp.where(valid, a * m16, 0.0)
    rows2 = lax.broadcasted_iota(jnp.int32, (BE8 // 2, 1), 0) + i * (BE8 // 2)
    valid2 = rows2 < E16
    m8r = jnp.dot(m16in_ref[...], ex2_ref[...],
                  preferred_element_type=jnp.float32,
                  precision=lax.Precision.HIGHEST)
    m8o_ref[...] = jnp.where(valid2, m8r, 0.0)


def _contrib_call(a8, m8in, m16in, exg, ex2):
    nb_real = E8 // BE8
    grid = E8_PAD // BE8
    full = lambda i: (0, 0)
    return pl.pallas_call(
        _contrib_body,
        grid=(grid,),
        in_specs=[
            pl.BlockSpec((BE8, 128), lambda i: (jnp.minimum(i, E8 // BE8 - 1), 0)),
            pl.BlockSpec((BE8, 8), lambda i: (jnp.minimum(i, E8 // BE8 - 1), 0)),
            pl.BlockSpec((BE8 // 2, 16), lambda i: (jnp.minimum(i, E8 // BE8 - 1), 0)),
            pl.BlockSpec((8, 128), full),
            pl.BlockSpec((16, 128), full),
        ],
        out_specs=[
            pl.BlockSpec((BE8, 128), lambda i: (i, 0)),
            pl.BlockSpec((BE8 // 2, 128), lambda i: (i, 0)),
        ],
        out_shape=[
            jax.ShapeDtypeStruct((E8_PAD, 128), jnp.float32),
            jax.ShapeDtypeStruct((E16_PAD, 128), jnp.float32),
        ],
    )(a8, m8in, m16in, exg, ex2)


def _scatter_attr_body(col2d, contrib, zeros, p0, p1, acc, contrib_v, idx_v,
                       sem):
    cid = lax.axis_index("c")
    sid = lax.axis_index("s")
    stage = contrib_v.at[pl.ds(0, N_STAGE)]

    for r in range(8):
        off = sid * N_PER_TILE + r * N_STAGE
        pltpu.sync_copy(zeros.at[pl.ds(off, N_STAGE)], stage)
        pltpu.sync_copy(stage, acc.at[pl.ds(off, N_STAGE)])
    plsc.subcore_barrier()

    base = cid * (16 * PER_W) + sid * PER_W

    def chunk(t, carry):
        r0 = base + t * SC_NK
        pltpu.sync_copy(col2d.at[pl.ds(r0, SC_NK)], idx_v)
        pltpu.sync_copy(contrib.at[pl.ds(r0 * 128, SC_NK * 128)], contrib_v)
        descs = []
        for j in range(SC_NK):
            descs.append(
                pltpu.async_copy(
                    contrib_v.at[pl.ds(j * 128, 128)],
                    acc.at[idx_v.at[j]],
                    sem,
                    add=True,
                ))
        for d in descs:
            d.wait()
        return carry

    lax.fori_loop(0, SC_CHUNKS, chunk, 0)
    plsc.subcore_barrier()

    for r in range(8):
        off = sid * N_PER_TILE + r * N_STAGE
        pltpu.sync_copy(acc.at[pl.ds(off, N_STAGE)], stage)

        @pl.when(cid == 0)
        def _():
            pltpu.sync_copy(stage, p0.at[pl.ds(off, N_STAGE)])

        @pl.when(cid == 1)
        def _():
            pltpu.sync_copy(stage, p1.at[pl.ds(off, N_STAGE)])


_scatter_attr_call = pl.kernel(
    _scatter_attr_body,
    out_type=[
        jax.ShapeDtypeStruct((N_PAD, D), jnp.float32),
        jax.ShapeDtypeStruct((N_PAD, D), jnp.float32),
    ],
    mesh=_SC_MESH,
    compiler_params=_SC_PARAMS,
    scratch_types=[
        pltpu.VMEM_SHARED((N_PAD, D), jnp.float32),
        pltpu.VMEM((SC_NK * 128, D), jnp.float32),
        pltpu.VMEM((SC_NK, 128), jnp.int32),
        pltpu.SemaphoreType.DMA,
    ],
)


MK = 16
M_CHUNKS = PER_W // MK


def _scatter_mask_body(col2d, mask2d, zeros, q0, q1, acc, mask_v, idx_v, sem):
    cid = lax.axis_index("c")
    sid = lax.axis_index("s")
    stage = mask_v.at[pl.ds(0, N_STAGE)]

    for r in range(8):
        off = sid * N_PER_TILE + r * N_STAGE
        pltpu.sync_copy(zeros.at[pl.ds(off, N_STAGE)], stage)
        pltpu.sync_copy(stage, acc.at[pl.ds(off, N_STAGE)])
    plsc.subcore_barrier()

    base = cid * (16 * PER_W) + sid * PER_W

    def chunk(t, carry):
        r0 = base + t * MK
        pltpu.sync_copy(col2d.at[pl.ds(r0, MK)], idx_v)
        pltpu.sync_copy(mask2d.at[pl.ds(r0 * 128, MK * 128)], mask_v)
        descs = []
        for j in range(MK):
            descs.append(
                pltpu.async_copy(
                    mask_v.at[pl.ds(j * 128, 128)],
                    acc.at[idx_v.at[j]],
                    sem,
                    add=True,
                ))
        for d in descs:
            d.wait()
        return carry

    lax.fori_loop(0, M_CHUNKS, chunk, 0)
    plsc.subcore_barrier()

    for r in range(8):
        off = sid * N_PER_TILE + r * N_STAGE
        pltpu.sync_copy(acc.at[pl.ds(off, N_STAGE)], stage)

        @pl.when(cid == 0)
        def _():
            pltpu.sync_copy(stage, q0.at[pl.ds(off, N_STAGE)])

        @pl.when(cid == 1)
        def _():
            pltpu.sync_copy(stage, q1.at[pl.ds(off, N_STAGE)])


_scatter_mask_call = pl.kernel(
    _scatter_mask_body,
    out_type=[
        jax.ShapeDtypeStruct((N_PAD, 8), jnp.float32),
        jax.ShapeDtypeStruct((N_PAD, 8), jnp.float32),
    ],
    mesh=_SC_MESH,
    compiler_params=_SC_PARAMS,
    scratch_types=[
        pltpu.VMEM_SHARED((N_PAD, 8), jnp.float32),
        pltpu.VMEM((MK * 128, 8), jnp.float32),
        pltpu.VMEM((MK, 128), jnp.int32),
        pltpu.SemaphoreType.DMA,
    ],
)


def _node_body(p0_ref, p1_ref, q0_ref, q1_ref, x_ref, wrt_ref, wct_ref,
               rep_ref, pr_ref, pc_ref):
    tot = p0_ref[...] + p1_ref[...]
    den = q0_ref[:, 0:1] + q1_ref[:, 0:1] + 1.0
    rep16 = tot / den
    rep = jnp.concatenate([rep16, x_ref[...]], axis=1)
    rep_ref[...] = rep
    pr_ref[...] = jnp.dot(rep, wrt_ref[...], preferred_element_type=jnp.float32,
                    precision=lax.Precision.HIGHEST)
    pc_ref[...] = jnp.dot(rep, wct_ref[...], preferred_element_type=jnp.float32,
                    precision=lax.Precision.HIGHEST)


def _node_call(p0, p1, q0, q1, x, wrt, wct):
    grid = N // BN
    full = lambda i: (0, 0)
    blk = lambda i: (i, 0)
    return pl.pallas_call(
        _node_body,
        grid=(grid,),
        in_specs=[
            pl.BlockSpec((BN, D), blk),
            pl.BlockSpec((BN, D), blk),
            pl.BlockSpec((BN, 8), blk),
            pl.BlockSpec((BN, 8), blk),
            pl.BlockSpec((BN, NODE_DIM), blk),
            pl.BlockSpec((D + NODE_DIM, D), full),
            pl.BlockSpec((D + NODE_DIM, D), full),
        ],
        out_specs=[
            pl.BlockSpec((BN, D + NODE_DIM), blk),
            pl.BlockSpec((BN, D), blk),
            pl.BlockSpec((BN, D), blk),
        ],
        out_shape=[
            jax.ShapeDtypeStruct((N, D + NODE_DIM), jnp.float32),
            jax.ShapeDtypeStruct((N, D), jnp.float32),
            jax.ShapeDtypeStruct((N, D), jnp.float32),
        ],
    )(p0, p1, q0, q1, x, wrt, wct)


def _gather_body(row2d, col2d, pr, pc, gr, gc, idx_r, idx_c, rows_r, rows_c,
                 sem):
    cid = lax.axis_index("c")
    sid = lax.axis_index("s")
    base = cid * (16 * PER_W) + sid * PER_W

    def chunk(t, carry):
        r0 = base + t * G_NK
        pltpu.sync_copy(row2d.at[pl.ds(r0, G_NK)], idx_r)
        pltpu.sync_copy(col2d.at[pl.ds(r0, G_NK)], idx_c)
        descs = []
        for j in range(G_NK):
            descs.append(
                pltpu.async_copy(pr.at[idx_r.at[j]],
                                 rows_r.at[pl.ds(j * 128, 128)], sem))
            descs.append(
                pltpu.async_copy(pc.at[idx_c.at[j]],
                                 rows_c.at[pl.ds(j * 128, 128)], sem))
        for d in descs:
            d.wait()
        pltpu.sync_copy(rows_r, gr.at[pl.ds(r0 * 128, G_NK * 128)])
        pltpu.sync_copy(rows_c, gc.at[pl.ds(r0 * 128, G_NK * 128)])
        return carry

    lax.fori_loop(0, G_CHUNKS, chunk, 0)


_gather_call = pl.kernel(
    _gather_body,
    out_type=[
        jax.ShapeDtypeStruct((E_PAD, D), jnp.float32),
        jax.ShapeDtypeStruct((E_PAD, D), jnp.float32),
    ],
    mesh=_SC_MESH,
    compiler_params=_SC_PARAMS,
    scratch_types=[
        pltpu.VMEM((G_NK, 128), jnp.int32),
        pltpu.VMEM((G_NK, 128), jnp.int32),
        pltpu.VMEM((G_NK * 128, D), jnp.float32),
        pltpu.VMEM((G_NK * 128, D), jnp.float32),
        pltpu.SemaphoreType.DMA,
    ],
)


def _mlp_body(a8_ref, gr8_ref, gc8_ref, watbd_ref, mmat_ref, w1bd_ref,
              w2bd_ref, be_ref, b1_ref, b2_ref, g1_ref, bb1_ref, g2_ref,
              bb2_ref, out_ref):
    dot = lambda x, w: jnp.dot(x, w, preferred_element_type=jnp.float32)
    a = a8_ref[...]
    h0 = dot(a, watbd_ref[...]) + gr8_ref[...] + gc8_ref[...] + be_ref[...]
    mm = mmat_ref[...]
    d1 = h0 - dot(h0, mm)
    var1 = dot(d1 * d1, mm)
    h = d1 * lax.rsqrt(var1 + 1e-5) * g1_ref[...] + bb1_ref[...]
    ff = jnp.maximum(dot(h, w1bd_ref[...]) + b1_ref[...], 0.0)
    s = h + dot(ff, w2bd_ref[...]) + b2_ref[...]
    d2 = s - dot(s, mm)
    var2 = dot(d2 * d2, mm)
    out_ref[...] = d2 * lax.rsqrt(var2 + 1e-5) * g2_ref[...] + bb2_ref[...]


def _mlp_call(a8, gr8, gc8, watbd, mmat, w1bd, w2bd, be, b1, b2, g1, bb1, g2,
              bb2):
    grid = E8 // BE8
    full = lambda i: (0, 0)
    blk = lambda i: (i, 0)
    return pl.pallas_call(
        _mlp_body,
        grid=(grid,),
        in_specs=[
            pl.BlockSpec((BE8, 128), blk),
            pl.BlockSpec((BE8, 128), blk),
            pl.BlockSpec((BE8, 128), blk),
            pl.BlockSpec((128, 128), full),
            pl.BlockSpec((128, 128), full),
            pl.BlockSpec((128, 256), full),
            pl.BlockSpec((256, 128), full),
            pl.BlockSpec((1, 128), full),
            pl.BlockSpec((1, 256), full),
            pl.BlockSpec((1, 128), full),
            pl.BlockSpec((1, 128), full),
            pl.BlockSpec((1, 128), full),
            pl.BlockSpec((1, 128), full),
            pl.BlockSpec((1, 128), full),
        ],
        out_specs=pl.BlockSpec((BE8, 128), blk),
        out_shape=jax.ShapeDtypeStruct((E8, 128), jnp.float32),
    )(a8, gr8, gc8, watbd, mmat, w1bd, w2bd, be, b1, b2, g1, bb1, g2, bb2)


def kernel(x, num_nodes, edge_index, edge_attr, mask, W_edge, b_edge, W1, b1,
           W2, b2, ln1_g, ln1_b, ln2_g, ln2_b):
    del num_nodes
    row = edge_index[0]
    col = edge_index[1]
    f32 = jnp.float32

    wrt = W_edge[:, :D + NODE_DIM].T
    wct = W_edge[:, D + NODE_DIM:2 * (D + NODE_DIM)].T
    wat = W_edge[:, 2 * (D + NODE_DIM):].T + jnp.eye(D, dtype=f32)
    eye8 = jnp.eye(8, dtype=f32)
    watbd = jnp.kron(eye8, wat)
    w1bd = jnp.kron(eye8, W1.T)
    w2bd = jnp.kron(eye8, W2.T)
    mmat = jnp.kron(eye8, jnp.full((D, D), 1.0 / D, f32))
    exg = jnp.kron(eye8, jnp.ones((1, D), f32))
    ex2 = jnp.kron(jnp.eye(D, dtype=f32), jnp.eye(8, dtype=f32)[0:1])
    tile8 = lambda v: jnp.tile(v, 8).reshape(1, -1)

    padv = lax.iota(jnp.int32, PAD)
    colp = jnp.concatenate([col, padv]).reshape(NROWS, 128)
    rowp = jnp.concatenate([row, padv]).reshape(NROWS, 128)

    a8 = edge_attr.reshape(E8, 128)
    contrib8, mask8 = _contrib_call(a8, mask.reshape(E8, 8),
                                    mask.reshape(E16, 16), exg, ex2)
    contrib = contrib8.reshape(E_PAD, D)
    zeros16 = jnp.zeros((N_PAD, D), f32)
    zeros8 = jnp.zeros((N_PAD, 8), f32)
    p0, p1 = _scatter_attr_call(colp, contrib, zeros16)
    q0, q1 = _scatter_mask_call(colp, mask8.reshape(E_PAD, 8), zeros8)
    node_rep, pr, pc = _node_call(p0, p1, q0, q1, x, wrt, wct)
    gr, gc = _gather_call(rowp, colp, pr, pc)
    out8 = _mlp_call(a8, gr.reshape(E8_PAD, 128),
                     gc.reshape(E8_PAD, 128), watbd, mmat, w1bd, w2bd,
                     tile8(b_edge), tile8(b1), tile8(b2), tile8(ln1_g),
                     tile8(ln1_b), tile8(ln2_g), tile8(ln2_b))
    return node_rep, out8.reshape(E, D)
